# baseline (speedup 1.0000x reference)
import jax
import jax.numpy as jnp
from jax.experimental import pallas as pl

_N = 10000
_E = 320000
_D = 128
_H = 8
_DH = 16
_S = 3
_SCALES = (50.0, 200.0, 500.0)
_BINS = 50


def _proj_body(x_ref, w_ref, b_ref, o_ref):
    o_ref[...] = (
        jnp.dot(x_ref[...], w_ref[...], preferred_element_type=jnp.float32)
        + b_ref[...]
    )


def _project(x, W, b):
    M, D = x.shape
    P = W.shape[1]
    BM = 1000
    return pl.pallas_call(
        _proj_body,
        grid=(M // BM,),
        in_specs=[
            pl.BlockSpec((BM, D), lambda i: (i, 0)),
            pl.BlockSpec((D, P), lambda i: (0, 0)),
            pl.BlockSpec((1, P), lambda i: (0, 0)),
        ],
        out_specs=pl.BlockSpec((BM, P), lambda i: (i, 0)),
        out_shape=jax.ShapeDtypeStruct((M, P), jnp.float32),
    )(x, W, b.reshape(1, P))


def kernel(x, edge_index, edge_attr, Wq, bq, Wk, bk, Wv, bv, Wo, bo, dist_emb, Wd, bd, scale_weights):
    src = edge_index[0]
    dst = edge_index[1]

    W_all = jnp.concatenate(
        [
            jnp.concatenate([Wq[s].T for s in range(_S)], axis=1),
            jnp.concatenate([Wk[s].T for s in range(_S)], axis=1),
            jnp.concatenate([Wv[s].T for s in range(_S)], axis=1),
        ],
        axis=1,
    )
    b_all = jnp.concatenate(
        [bq.reshape(-1), bk.reshape(-1), bv.reshape(-1)], axis=0
    )
    qkv = _project(x, W_all, b_all)
    q_all = qkv[:, 0 * _S * _D:1 * _S * _D]
    k_all = qkv[:, 1 * _S * _D:2 * _S * _D]
    v_all = qkv[:, 2 * _S * _D:3 * _S * _D]

    outs = []
    for s in range(_S):
        q = q_all[:, s * _D:(s + 1) * _D].reshape(_N, _H, _DH)
        k = k_all[:, s * _D:(s + 1) * _D].reshape(_N, _H, _DH)
        v = v_all[:, s * _D:(s + 1) * _D].reshape(_N, _H, _DH)
        q_i = q[dst]
        k_j = k[src]
        v_j = v[src]
        scores = (q_i * k_j).sum(axis=-1) * (_DH ** -0.5)
        distances = edge_attr[:, 0]
        bins = jnp.clip((distances / _SCALES[s] * _BINS).astype(jnp.int32), 0, _BINS)
        bias = dist_emb[s][bins]
        dirs = edge_attr[:, 1:3]
        nrm = jnp.clip(jnp.linalg.norm(dirs, axis=1, keepdims=True), 1e-8, None)
        nd = dirs / nrm
        bias = bias + jnp.tanh(nd @ Wd[s].T + bd[s])
        scores = scores + bias
        smax = jax.ops.segment_max(scores, dst, num_segments=_N)
        smax = jnp.where(jnp.isfinite(smax), smax, 0.0)
        ex = jnp.exp(scores - smax[dst])
        denom = jax.ops.segment_sum(ex, dst, num_segments=_N)
        attn = ex / (denom[dst] + 1e-16)
        msg = attn[:, :, None] * v_j
        aggr = jax.ops.segment_sum(msg, dst, num_segments=_N)
        out = _project(aggr.reshape(_N, _D), Wo[s].T, bo[s])
        outs.append(out)

    w = jax.nn.softmax(scale_weights)
    out = w[0] * outs[0]
    for s in range(1, _S):
        out = out + w[s] * outs[s]
    return out


# speedup vs baseline: 12.0106x; 12.0106x over previous
import dataclasses
import functools

import jax
import jax.numpy as jnp
from jax import lax
from jax.experimental import pallas as pl
from jax.experimental.pallas import tpu as pltpu
from jax.experimental.pallas import tpu_sc as plsc

_N = 10000
_E = 320000
_D = 128
_H = 8
_DH = 16
_S = 3
_SCALES = (50.0, 200.0, 500.0)
_BINS = 50

_NW = 32
_B = 128
_NCHUNK_TOT = _E // _B
_NT = 624
_NTAIL = _N - 16 * _NT


def _proj_body(x_ref, w_ref, b_ref, *o_refs):
    acc = jnp.dot(x_ref[...], w_ref[...], preferred_element_type=jnp.float32)
    acc = acc + b_ref[...]
    for i, o_ref in enumerate(o_refs):
        o_ref[...] = acc[:, i * _D:(i + 1) * _D]


def _project_qkv(x, W_all, b_all):
    BM = 1000
    P = 9 * _D
    outs = [jax.ShapeDtypeStruct((_N, _D), jnp.float32) for _ in range(9)]
    return pl.pallas_call(
        _proj_body,
        grid=(_N // BM,),
        in_specs=[
            pl.BlockSpec((BM, _D), lambda i: (i, 0)),
            pl.BlockSpec((_D, P), lambda i: (0, 0)),
            pl.BlockSpec((1, P), lambda i: (0, 0)),
        ],
        out_specs=[pl.BlockSpec((BM, _D), lambda i: (i, 0)) for _ in range(9)],
        out_shape=outs,
    )(x, W_all, b_all.reshape(1, P))


def _bias_body(ea_ref, dembT_ref, wd_ref, bd_ref, o_ref):
    ea = ea_ref[...]
    dist = ea[0:1, :]
    dx = ea[1:2, :]
    dy = ea[2:3, :]
    nrm = jnp.maximum(jnp.sqrt(dx * dx + dy * dy), 1e-8)
    ndx = dx / nrm
    ndy = dy / nrm
    BE = ea.shape[1]
    row = jax.lax.broadcasted_iota(jnp.int32, (_BINS + 1, BE), 0)
    for s in range(_S):
        bins = jnp.clip((dist * (_BINS / _SCALES[s])).astype(jnp.int32), 0, _BINS)
        oh = (row == bins).astype(jnp.float32)
        bias_d = jnp.dot(dembT_ref[s], oh, preferred_element_type=jnp.float32)
        wd = wd_ref[s]
        targ = wd[:, 0:1] * ndx + wd[:, 1:2] * ndy + bd_ref[s].reshape(_H, 1)
        o_ref[s] = bias_d + jnp.tanh(targ)


def _edge_bias(edge_attrT, dist_embT, Wd, bd):
    BE = 3200
    return pl.pallas_call(
        _bias_body,
        grid=(_E // BE,),
        in_specs=[
            pl.BlockSpec((4, BE), lambda i: (0, i)),
            pl.BlockSpec((_S, _H, _BINS + 1), lambda i: (0, 0, 0)),
            pl.BlockSpec((_S, _H, 2), lambda i: (0, 0, 0)),
            pl.BlockSpec((_S, _H), lambda i: (0, 0)),
        ],
        out_specs=pl.BlockSpec((_S, _H, BE), lambda i: (0, 0, i)),
        out_shape=jax.ShapeDtypeStruct((_S, _H, _E), jnp.float32),
    )(edge_attrT, dist_embT, Wd, bd)


def _edge_body(q0, q1, q2, k0, k1, k2, v0, v1, v2, src_hbm, dst_hbm,
               b0, b1, b2,
               aggr_out, denom_out,
               src_v, dst_v, q_rows, k_q, bias_v,
               ex_rows, aggr_sh, denom_sh, sem0, sem1):
    c = lax.axis_index("c")
    t = lax.axis_index("s")
    wid = c * 16 + t
    off = t * _NT
    lanes = lax.iota(jnp.int32, 16)
    nfull = _NCHUNK_TOT // _NW
    nchunks = nfull + jnp.where(wid < _NCHUNK_TOT - nfull * _NW, 1, 0)
    zero16 = jnp.zeros((16,), jnp.float32)

    for s in range(_S):
        q_hbm = (q0, q1, q2)[s]
        k_hbm = (k0, k1, k2)[s]
        v_hbm = (v0, v1, v2)[s]
        biasT_hbm = (b0, b1, b2)[s]

        def _zq(e, carry):
            ex_rows[e, :] = zero16
            for c8 in range(8):
                q_rows[e, pl.ds(c8 * 16, 16)] = zero16
            return carry
        lax.fori_loop(0, _B, _zq, 0)

        for j in range(4):
            pltpu.sync_copy(q_rows, aggr_sh.at[pl.ds(off + j * 128, 128)])
            pltpu.sync_copy(ex_rows, denom_sh.at[pl.ds(off + j * 128, 128)])
        pltpu.sync_copy(q_rows.at[pl.ds(0, 112)],
                        aggr_sh.at[pl.ds(off + 512, 112)])
        pltpu.sync_copy(ex_rows.at[pl.ds(0, 112)],
                        denom_sh.at[pl.ds(off + 512, 112)])

        @pl.when(t == 15)
        def _zero_tail():
            pltpu.sync_copy(q_rows.at[pl.ds(0, _NTAIL)],
                            aggr_sh.at[pl.ds(16 * _NT, _NTAIL)])
            pltpu.sync_copy(ex_rows.at[pl.ds(0, _NTAIL)],
                            denom_sh.at[pl.ds(16 * _NT, _NTAIL)])

        plsc.subcore_barrier()

        def chunk_body(k, carry):
            base = (wid + k * _NW) * _B
            pltpu.sync_copy(src_hbm.at[pl.ds(base, _B)], src_v)
            pltpu.sync_copy(dst_hbm.at[pl.ds(base, _B)], dst_v)
            pltpu.sync_copy(biasT_hbm.at[:, pl.ds(base, _B)], bias_v)

            pltpu.async_copy(q_hbm.at[dst_v], q_rows, sem0).wait()

            for quarter in range(4):
                qb = quarter * 32
                pltpu.async_copy(
                    k_hbm.at[src_v.at[pl.ds(qb, 32)]], k_q, sem1).wait()

                def score_body(j, carry2):
                    rows16 = j * 16 + lanes
                    qrows16 = qb + rows16
                    for h in range(_H):
                        colbase = h * 16
                        score = jnp.zeros((16,), jnp.float32)
                        for d in range(_DH):
                            colv = jnp.full((16,), colbase + d, jnp.int32)
                            qv = plsc.load_gather(q_rows, [qrows16, colv])
                            kv = plsc.load_gather(k_q, [rows16, colv])
                            score = score + qv * kv
                        bias16 = bias_v[h, pl.ds(qb + j * 16, 16)]
                        ex = jnp.exp(score + bias16)
                        plsc.store_scatter(
                            ex_rows,
                            [qrows16, jnp.full((16,), h, jnp.int32)], ex)
                    return carry2

                lax.fori_loop(0, 2, score_body, 0)

            pltpu.async_copy(v_hbm.at[src_v], q_rows, sem0).wait()

            def msg_body(j, carry2):
                rows16 = j * 16 + lanes
                for h in range(_H):
                    colbase = h * 16
                    ex = plsc.load_gather(
                        ex_rows, [rows16, jnp.full((16,), h, jnp.int32)])
                    for d in range(_DH):
                        colv = jnp.full((16,), colbase + d, jnp.int32)
                        vv = plsc.load_gather(q_rows, [rows16, colv])
                        plsc.store_scatter(q_rows, [rows16, colv], vv * ex)
                return carry2

            lax.fori_loop(0, _B // 16, msg_body, 0)
            pltpu.sync_copy(q_rows, aggr_sh.at[dst_v], add=True)
            pltpu.sync_copy(ex_rows, denom_sh.at[dst_v], add=True)
            return carry

        lax.fori_loop(0, nchunks, chunk_body, 0)
        plsc.subcore_barrier()

        pltpu.sync_copy(aggr_sh.at[pl.ds(off, _NT)],
                        aggr_out.at[s, c, pl.ds(off, _NT)])
        pltpu.sync_copy(denom_sh.at[pl.ds(off, _NT)],
                        denom_out.at[s, c, pl.ds(off, _NT)])

        @pl.when(t == 15)
        def _drain_tail():
            pltpu.sync_copy(aggr_sh.at[pl.ds(16 * _NT, _NTAIL)],
                            aggr_out.at[s, c, pl.ds(16 * _NT, _NTAIL)])
            pltpu.sync_copy(denom_sh.at[pl.ds(16 * _NT, _NTAIL)],
                            denom_out.at[s, c, pl.ds(16 * _NT, _NTAIL)])


def _make_edge_kernel():
    mesh = plsc.VectorSubcoreMesh(core_axis_name="c", subcore_axis_name="s")
    cp = pltpu.CompilerParams(use_tc_tiling_on_sc=False)
    if "needs_layout_passes" in pltpu.CompilerParams.__dataclass_fields__:
        cp = dataclasses.replace(cp, needs_layout_passes=False)
    return pl.kernel(
        _edge_body,
        compiler_params=cp,
        out_type=[
            jax.ShapeDtypeStruct((_S, 2, _N, _D), jnp.float32),
            jax.ShapeDtypeStruct((_S, 2, _N, 16), jnp.float32),
        ],
        mesh=mesh,
        scratch_types=[
            pltpu.VMEM((_B,), jnp.int32),
            pltpu.VMEM((_B,), jnp.int32),
            pltpu.VMEM((_B, _D), jnp.float32),
            pltpu.VMEM((32, _D), jnp.float32),
            pltpu.VMEM((_H, _B), jnp.float32),
            pltpu.VMEM((_B, 16), jnp.float32),
            pltpu.VMEM_SHARED((_N, _D), jnp.float32),
            pltpu.VMEM_SHARED((_N, 16), jnp.float32),
            pltpu.SemaphoreType.DMA,
            pltpu.SemaphoreType.DMA,
        ],
    )


def _combine_body(aggr_ref, denom_ref, wcat_ref, bcomb_ref, o_ref):
    parts = []
    for s in range(_S):
        A = aggr_ref[s, 0] + aggr_ref[s, 1]
        dn = denom_ref[s, 0, :, :_H] + denom_ref[s, 1, :, :_H]
        r = 1.0 / (dn + 1e-16)
        cols = [A[:, h * 16:(h + 1) * 16] * r[:, h:h + 1] for h in range(_H)]
        parts.append(jnp.concatenate(cols, axis=1))
    An = jnp.concatenate(parts, axis=1)
    o_ref[...] = (
        jnp.dot(An, wcat_ref[...], preferred_element_type=jnp.float32)
        + bcomb_ref[...]
    )


def _combine(aggr, denom, Wcat, bcomb):
    BM = 1000
    return pl.pallas_call(
        _combine_body,
        grid=(_N // BM,),
        in_specs=[
            pl.BlockSpec((_S, 2, BM, _D), lambda i: (0, 0, i, 0)),
            pl.BlockSpec((_S, 2, BM, 16), lambda i: (0, 0, i, 0)),
            pl.BlockSpec((_S * _D, _D), lambda i: (0, 0)),
            pl.BlockSpec((1, _D), lambda i: (0, 0)),
        ],
        out_specs=pl.BlockSpec((BM, _D), lambda i: (i, 0)),
        out_shape=jax.ShapeDtypeStruct((_N, _D), jnp.float32),
    )(aggr, denom, Wcat, bcomb.reshape(1, _D))


def kernel(x, edge_index, edge_attr, Wq, bq, Wk, bk, Wv, bv, Wo, bo, dist_emb, Wd, bd, scale_weights):
    scale = _DH ** -0.5
    W_all = jnp.concatenate(
        [jnp.concatenate([Wq[s].T * scale for s in range(_S)], axis=1),
         jnp.concatenate([Wk[s].T for s in range(_S)], axis=1),
         jnp.concatenate([Wv[s].T for s in range(_S)], axis=1)],
        axis=1,
    )
    b_all = jnp.concatenate([bq.reshape(-1) * scale, bk.reshape(-1),
                             bv.reshape(-1)], axis=0)
    tabs = _project_qkv(x, W_all, b_all)
    q_tabs, k_tabs, v_tabs = tabs[0:3], tabs[3:6], tabs[6:9]

    biasT = _edge_bias(edge_attr.T, jnp.transpose(dist_emb, (0, 2, 1)), Wd, bd)

    src = edge_index[0]
    dst = edge_index[1]

    edge_kernel = _make_edge_kernel()
    aggr, denom = edge_kernel(
        q_tabs[0], q_tabs[1], q_tabs[2],
        k_tabs[0], k_tabs[1], k_tabs[2],
        v_tabs[0], v_tabs[1], v_tabs[2],
        src, dst, biasT[0], biasT[1], biasT[2])

    w = jax.nn.softmax(scale_weights)
    Wcat = jnp.concatenate([w[s] * Wo[s].T for s in range(_S)], axis=0)
    bcomb = (w[:, None] * bo).sum(axis=0)
    return _combine(aggr, denom, Wcat, bcomb)
